# baseline (speedup 1.0000x reference)
import functools

import jax
import jax.numpy as jnp
from jax import lax
from jax.experimental import pallas as pl
from jax.experimental.pallas import tpu as pltpu
from jax.experimental.pallas import tpu_sc as plsc

NC = 2
NS = 16
NW = NC * NS
CHUNK = 128


def _sc_gather4(user, item, mf_user_emb, mf_item_emb, mlp_user_emb, mlp_item_emb):
    B = user.shape[0]
    bpw = B // NW
    nch = bpw // CHUNK
    dmf = mf_user_emb.shape[1]
    dml = mlp_user_emb.shape[1]
    mesh = plsc.VectorSubcoreMesh(core_axis_name="c", subcore_axis_name="s")

    @functools.partial(
        pl.kernel,
        mesh=mesh,
        compiler_params=pltpu.CompilerParams(use_tc_tiling_on_sc=False),
        out_type=[
            jax.ShapeDtypeStruct((B, dmf), jnp.float32),
            jax.ShapeDtypeStruct((B, dmf), jnp.float32),
            jax.ShapeDtypeStruct((B, dml), jnp.float32),
            jax.ShapeDtypeStruct((B, dml), jnp.float32),
        ],
        scratch_types=[
            pltpu.VMEM((nch, CHUNK), jnp.int32),
            pltpu.VMEM((nch, CHUNK), jnp.int32),
            pltpu.VMEM((bpw, dmf), jnp.float32),
            pltpu.VMEM((bpw, dmf), jnp.float32),
            pltpu.VMEM((bpw, dml), jnp.float32),
            pltpu.VMEM((bpw, dml), jnp.float32),
            pltpu.SemaphoreType.DMA,
        ],
    )
    def k(user_hbm, item_hbm, mfu_hbm, mfi_hbm, mlu_hbm, mli_hbm,
          out_mfu, out_mfi, out_mlu, out_mli,
          uidx, iidx, bmfu, bmfi, bmlu, bmli, sem):
        wid = lax.axis_index("s") * NC + lax.axis_index("c")
        base = wid * bpw
        for j in range(nch):
            pltpu.sync_copy(user_hbm.at[pl.ds(base + j * CHUNK, CHUNK)], uidx.at[j])
            pltpu.sync_copy(item_hbm.at[pl.ds(base + j * CHUNK, CHUNK)], iidx.at[j])
        copies = []
        for j in range(nch):
            s = pl.ds(j * CHUNK, CHUNK)
            copies.append(pltpu.async_copy(mfu_hbm.at[uidx.at[j]], bmfu.at[s], sem))
            copies.append(pltpu.async_copy(mfi_hbm.at[iidx.at[j]], bmfi.at[s], sem))
            copies.append(pltpu.async_copy(mlu_hbm.at[uidx.at[j]], bmlu.at[s], sem))
            copies.append(pltpu.async_copy(mli_hbm.at[iidx.at[j]], bmli.at[s], sem))
        for c in copies:
            c.wait()
        dst = pl.ds(base, bpw)
        pltpu.sync_copy(bmfu, out_mfu.at[dst])
        pltpu.sync_copy(bmfi, out_mfi.at[dst])
        pltpu.sync_copy(bmlu, out_mlu.at[dst])
        pltpu.sync_copy(bmli, out_mli.at[dst])

    return k(user, item, mf_user_emb, mf_item_emb, mlp_user_emb, mlp_item_emb)


def _tc_body(mfu_ref, mfi_ref, mlu_ref, mli_ref, w0u_ref, w0i_ref, b0_ref,
             w1_ref, b1_ref, w2_ref, b2_ref, wpm_ref, wph_ref, bp_ref, out_ref):
    f32 = jnp.float32
    h = jnp.dot(mlu_ref[...], w0u_ref[...], preferred_element_type=f32)
    h = h + jnp.dot(mli_ref[...], w0i_ref[...], preferred_element_type=f32)
    h = jnp.maximum(h + b0_ref[...], 0.0)
    h = jnp.maximum(jnp.dot(h, w1_ref[...], preferred_element_type=f32) + b1_ref[...], 0.0)
    h = jnp.maximum(jnp.dot(h, w2_ref[...], preferred_element_type=f32) + b2_ref[...], 0.0)
    mf = mfu_ref[...] * mfi_ref[...]
    logit = (jnp.dot(mf, wpm_ref[...], preferred_element_type=f32)
             + jnp.dot(h, wph_ref[...], preferred_element_type=f32)
             + bp_ref[...])
    out_ref[...] = jax.nn.sigmoid(logit) * 4.5 + 0.5


def kernel(user, item, mf_user_emb, mf_item_emb, mlp_user_emb, mlp_item_emb,
           w0, b0, w1, b1, w2, b2, wp, bp):
    B = user.shape[0]
    dmf = mf_user_emb.shape[1]
    dml = mlp_user_emb.shape[1]
    mfu, mfi, mlu, mli = _sc_gather4(
        user, item, mf_user_emb, mf_item_emb, mlp_user_emb, mlp_item_emb)

    w0u = w0[:, :dml].T
    w0i = w0[:, dml:].T
    w1t = w1.T
    w2t = w2.T
    wpm = wp[:, :dmf].T
    wph = wp[:, dmf:].T
    b0r = b0.reshape(1, -1)
    b1r = b1.reshape(1, -1)
    b2r = b2.reshape(1, -1)
    bpr = bp.reshape(1, 1)

    R = 2048
    grid = (B // R,)
    d0 = w0.shape[0]
    d1 = w1.shape[0]
    d2 = w2.shape[0]
    data = lambda r, c: pl.BlockSpec((R, c), lambda i: (i, 0))
    full = lambda a, b: pl.BlockSpec((a, b), lambda i: (0, 0))
    out2 = pl.pallas_call(
        _tc_body,
        grid=grid,
        in_specs=[
            data(R, dmf), data(R, dmf), data(R, dml), data(R, dml),
            full(dml, d0), full(dml, d0), full(1, d0),
            full(d0, d1), full(1, d1),
            full(d1, d2), full(1, d2),
            full(dmf, 1), full(d2, 1), full(1, 1),
        ],
        out_specs=pl.BlockSpec((R, 1), lambda i: (i, 0)),
        out_shape=jax.ShapeDtypeStruct((B, 1), jnp.float32),
    )(mfu, mfi, mlu, mli, w0u, w0i, b0r, w1t, b1r, w2t, b2r, wpm, wph, bpr)
    return out2.reshape(B)
